# Initial kernel scaffold; baseline (speedup 1.0000x reference)
import functools

import jax
import jax.numpy as jnp
from jax import lax
from jax.experimental import pallas as pl
from jax.experimental.pallas import tpu as pltpu
from jax.experimental.pallas import tpu_sc as plsc

V = 50000
NROWS = 442368
FDIM = 128
NC, NS = 2, 16
NW = NC * NS
S = NROWS // NW
ZR = 256
NZ = S // ZR
CH = 2000
NCH = V // CH
K = 128
NPAD = 8 * NW
L = 16


def _body(cidx_hbm, rid_hbm, table_hbm, out_hbm,
          zbuf, winner, cchunk, rchunk, cbuf, rbuf, rowbuf,
          sem_z, sem_g, sem_s):
    wid = lax.axis_index("s") * NC + lax.axis_index("c")
    base = wid * S
    dump = NROWS + wid * 8
    iota = lax.iota(jnp.int32, L)
    zeros16 = jnp.zeros((L,), jnp.float32)

    def zrow(i, _):
        for k in range(FDIM // L):
            zbuf[i, pl.ds(k * L, L)] = zeros16
        return 0
    lax.fori_loop(0, ZR, zrow, 0)

    def zfire(k, _):
        pltpu.async_copy(zbuf, out_hbm.at[pl.ds(base + k * ZR, ZR)], sem_z)
        return 0
    lax.fori_loop(0, NZ, zfire, 0)

    neg1 = jnp.full((L,), -1, jnp.int32)
    def winit(i, _):
        winner[pl.ds(i * L, L)] = neg1
        return 0
    lax.fori_loop(0, S // L, winit, 0)

    def p1_chunk(t, _):
        c0 = t * CH
        pltpu.sync_copy(cidx_hbm.at[pl.ds(c0, CH)], cchunk)
        def p1_inner(j, _):
            c = cchunk[pl.ds(j * L, L)]
            v = c0 + j * L + iota
            crel = c - base
            m = (crel >= 0) & (crel < S)
            plsc.store_scatter(winner, [jnp.clip(crel, 0, S - 1)], v, mask=m)
            return 0
        lax.fori_loop(0, CH // L, p1_inner, 0)
        return 0
    lax.fori_loop(0, NCH, p1_chunk, 0)

    dump16 = dump + jnp.zeros((L,), jnp.int32)
    zero16i = jnp.zeros((L,), jnp.int32)
    def binit(i, _):
        cbuf[pl.ds(i * L, L)] = dump16
        rbuf[pl.ds(i * L, L)] = zero16i
        return 0
    lax.fori_loop(0, K // L, binit, 0)

    def zdrain(k, _):
        pltpu.make_async_copy(zbuf, out_hbm.at[pl.ds(base + k * ZR, ZR)],
                              sem_z).wait()
        return 0
    lax.fori_loop(0, NZ, zdrain, 0)

    def fire():
        pltpu.async_copy(table_hbm.at[rbuf], rowbuf, sem_g).wait()
        pltpu.async_copy(rowbuf, out_hbm.at[cbuf], sem_s).wait()

    def p2_chunk(t, cnt):
        c0 = t * CH
        pltpu.sync_copy(cidx_hbm.at[pl.ds(c0, CH)], cchunk)
        pltpu.sync_copy(rid_hbm.at[pl.ds(c0, CH)], rchunk)
        def p2_inner(j, cnt):
            c = cchunk[pl.ds(j * L, L)]
            r = rchunk[pl.ds(j * L, L)]
            v = c0 + j * L + iota
            crel = c - base
            m1 = (crel >= 0) & (crel < S)
            w = plsc.load_gather(winner, [jnp.clip(crel, 0, S - 1)], mask=m1)
            m = m1 & (w == v)
            plsc.store_compressed(cbuf.at[pl.ds(cnt, L)], c, mask=m)
            plsc.store_compressed(rbuf.at[pl.ds(cnt, L)], r, mask=m)
            cnt = cnt + jnp.sum(m.astype(jnp.int32))
            do_fire = cnt > K - L
            @pl.when(do_fire)
            def _():
                fire()
            return jnp.where(do_fire, 0, cnt)
        return lax.fori_loop(0, CH // L, p2_inner, cnt)
    cnt = lax.fori_loop(0, NCH, p2_chunk, 0)

    @pl.when(cnt > 0)
    def _():
        fire()


@jax.jit
def _voxel_grid(cidx, rid, table):
    mesh = plsc.VectorSubcoreMesh(core_axis_name="c", subcore_axis_name="s",
                                  num_cores=NC, num_subcores=NS)
    f = pl.kernel(
        _body,
        out_type=jax.ShapeDtypeStruct((NROWS + NPAD, FDIM), jnp.float32),
        mesh=mesh,
        scratch_types=[
            pltpu.VMEM((ZR, FDIM), jnp.float32),
            pltpu.VMEM((S,), jnp.int32),
            pltpu.VMEM((CH,), jnp.int32),
            pltpu.VMEM((CH,), jnp.int32),
            pltpu.VMEM((K,), jnp.int32),
            pltpu.VMEM((K,), jnp.int32),
            pltpu.VMEM((K, FDIM), jnp.float32),
            pltpu.SemaphoreType.DMA,
            pltpu.SemaphoreType.DMA,
            pltpu.SemaphoreType.DMA,
        ],
    )
    return f(cidx, rid, table)


def kernel(voxels_argmax_centerIdx, voxels_argmax_batchResIds0Based,
           prot_feats0based, prot_multizProfiles, voxelFeats_proteinBatch):
    orig_shape = voxelFeats_proteinBatch.shape
    cidx = voxels_argmax_centerIdx.astype(jnp.int32)
    rid = voxels_argmax_batchResIds0Based.astype(jnp.int32)
    table = jnp.concatenate([prot_feats0based, prot_multizProfiles], axis=1)
    out = _voxel_grid(cidx, rid, table)
    return out[:NROWS].reshape(orig_shape)


# speedup vs baseline: 259.7174x; 259.7174x over previous
import functools

import jax
import jax.numpy as jnp
from jax import lax
from jax.experimental import pallas as pl
from jax.experimental.pallas import tpu as pltpu
from jax.experimental.pallas import tpu_sc as plsc

V = 50000
NROWS = 442368
FDIM = 128
NC, NS = 2, 16
NW = NC * NS
S = NROWS // NW
ZR = 256
NZ = S // ZR
CH = 2000
NCH = V // CH
K = 128
NPAD = 8 * NW
L = 16


def _body(cidx_hbm, rid_hbm, table_hbm, out_hbm,
          zbuf, winner, cchunk, rchunk, cbuf, rbuf, rowbuf,
          sem_z, sem_g, sem_s):
    wid = lax.axis_index("s") * NC + lax.axis_index("c")
    base = wid * S
    dump = NROWS + wid * 8
    iota = lax.iota(jnp.int32, L)
    zeros16 = jnp.zeros((L,), jnp.float32)

    def zrow(i, _):
        for k in range(FDIM // L):
            zbuf[i, pl.ds(k * L, L)] = zeros16
        return 0
    lax.fori_loop(0, ZR, zrow, 0)

    def zfire(k, _):
        pltpu.async_copy(zbuf, out_hbm.at[pl.ds(base + k * ZR, ZR)], sem_z)
        return 0
    lax.fori_loop(0, NZ, zfire, 0)

    neg1 = jnp.full((L,), -1, jnp.int32)
    def winit(i, _):
        winner[pl.ds(i * L, L)] = neg1
        return 0
    lax.fori_loop(0, S // L, winit, 0)

    def p1_chunk(t, _):
        c0 = t * CH
        pltpu.sync_copy(cidx_hbm.at[pl.ds(c0, CH)], cchunk)
        def p1_inner(j, _):
            c = cchunk[pl.ds(j * L, L)]
            v = c0 + j * L + iota
            crel = c - base
            m = (crel >= 0) & (crel < S)
            plsc.store_scatter(winner, [jnp.clip(crel, 0, S - 1)], v, mask=m)
            return 0
        lax.fori_loop(0, CH // L, p1_inner, 0)
        return 0
    lax.fori_loop(0, NCH, p1_chunk, 0)

    dump16 = dump + jnp.zeros((L,), jnp.int32)
    zero16i = jnp.zeros((L,), jnp.int32)
    def binit(i, _):
        cbuf[pl.ds(i * L, L)] = dump16
        rbuf[pl.ds(i * L, L)] = zero16i
        return 0
    lax.fori_loop(0, K // L, binit, 0)

    def zdrain(k, _):
        pltpu.make_async_copy(zbuf, out_hbm.at[pl.ds(base + k * ZR, ZR)],
                              sem_z).wait()
        return 0
    lax.fori_loop(0, NZ, zdrain, 0)

    def fire():
        pltpu.async_copy(table_hbm.at[rbuf], rowbuf, sem_g).wait()
        pltpu.async_copy(rowbuf, out_hbm.at[cbuf], sem_s).wait()

    def p2_chunk(t, cnt):
        c0 = t * CH
        pltpu.sync_copy(cidx_hbm.at[pl.ds(c0, CH)], cchunk)
        pltpu.sync_copy(rid_hbm.at[pl.ds(c0, CH)], rchunk)
        def p2_inner(j, cnt):
            c = cchunk[pl.ds(j * L, L)]
            r = rchunk[pl.ds(j * L, L)]
            v = c0 + j * L + iota
            crel = c - base
            m1 = (crel >= 0) & (crel < S)
            w = plsc.load_gather(winner, [jnp.clip(crel, 0, S - 1)], mask=m1)
            m = m1 & (w == v)
            plsc.store_compressed(cbuf.at[pl.ds(cnt, L)], c, mask=m)
            plsc.store_compressed(rbuf.at[pl.ds(cnt, L)], r, mask=m)
            cnt = cnt + jnp.sum(m.astype(jnp.int32))
            do_fire = cnt > K - L
            @pl.when(do_fire)
            def _():
                fire()
            return jnp.where(do_fire, 0, cnt)
        return lax.fori_loop(0, CH // L, p2_inner, cnt)
    cnt = lax.fori_loop(0, NCH, p2_chunk, 0)

    @pl.when(cnt > 0)
    def _():
        fire()


@jax.jit
def _voxel_grid(cidx, rid, table):
    mesh = plsc.VectorSubcoreMesh(core_axis_name="c", subcore_axis_name="s",
                                  num_cores=NC, num_subcores=NS)
    f = pl.kernel(
        _body,
        out_type=jax.ShapeDtypeStruct((NROWS + NPAD, FDIM), jnp.float32),
        mesh=mesh,
        compiler_params=pltpu.CompilerParams(needs_layout_passes=False),
        scratch_types=[
            pltpu.VMEM((ZR, FDIM), jnp.float32),
            pltpu.VMEM((S,), jnp.int32),
            pltpu.VMEM((CH,), jnp.int32),
            pltpu.VMEM((CH,), jnp.int32),
            pltpu.VMEM((K,), jnp.int32),
            pltpu.VMEM((K,), jnp.int32),
            pltpu.VMEM((K, FDIM), jnp.float32),
            pltpu.SemaphoreType.DMA,
            pltpu.SemaphoreType.DMA,
            pltpu.SemaphoreType.DMA,
        ],
    )
    return f(cidx, rid, table)


def kernel(voxels_argmax_centerIdx, voxels_argmax_batchResIds0Based,
           prot_feats0based, prot_multizProfiles, voxelFeats_proteinBatch):
    orig_shape = voxelFeats_proteinBatch.shape
    cidx = voxels_argmax_centerIdx.astype(jnp.int32)
    rid = voxels_argmax_batchResIds0Based.astype(jnp.int32)
    table = jnp.concatenate([prot_feats0based, prot_multizProfiles], axis=1)
    out = _voxel_grid(cidx, rid, table)
    return out[:NROWS].reshape(orig_shape)


# speedup vs baseline: 624.8905x; 2.4060x over previous
import jax
import jax.numpy as jnp
from jax import lax
from jax.experimental import pallas as pl
from jax.experimental.pallas import tpu as pltpu
from jax.experimental.pallas import tpu_sc as plsc

V = 50000
NROWS = 442368
FDIM = 128
NC, NS = 2, 16
NW = NC * NS
S = NROWS // NW
ZR = 216
NZ = S // ZR
CH = 2000
NCH = V // CH
NPAIR = (NCH + 1) // 2
K = 128
NFMAX = S // K
L = 16
RPAD = 4096
RSHIFT = 12


def _body(cidx_hbm, rid_hbm, table_hbm, out_hbm,
          zbuf, winner, cbig, rbig, cca, cra, ccb, crb,
          cb0, rb0, cb1, rb1, row0, row1,
          sem_z, sem_ca, sem_cb, sem_g0, sem_g1, sem_s0, sem_s1):
    wid = lax.axis_index("s") * NC + lax.axis_index("c")
    base = wid * S
    iota = lax.iota(jnp.int32, L)
    zeros16 = jnp.zeros((L,), jnp.float32)

    def zrow(i, _):
        for k in range(FDIM // L):
            zbuf[i, pl.ds(k * L, L)] = zeros16
        return 0
    lax.fori_loop(0, ZR, zrow, 0)

    def zfire(k, _):
        pltpu.async_copy(zbuf, out_hbm.at[pl.ds(base + k * ZR, ZR)], sem_z)
        return 0
    lax.fori_loop(0, NZ, zfire, 0)

    neg1 = jnp.full((L,), -1, jnp.int32)
    def winit(i, _):
        winner[pl.ds(i * L, L)] = neg1
        return 0
    lax.fori_loop(0, S // L, winit, 0)

    def cstart(t, cc, cr, sem):
        pltpu.async_copy(cidx_hbm.at[pl.ds(t * CH, CH)], cc, sem)
        pltpu.async_copy(rid_hbm.at[pl.ds(t * CH, CH)], cr, sem)

    def cwait(t, cc, cr, sem):
        pltpu.make_async_copy(cidx_hbm.at[pl.ds(t * CH, CH)], cc, sem).wait()
        pltpu.make_async_copy(rid_hbm.at[pl.ds(t * CH, CH)], cr, sem).wait()

    def process(t, cc, cr):
        c0 = t * CH
        def inner(j, _):
            c = cc[pl.ds(j * L, L)]
            r = cr[pl.ds(j * L, L)]
            v = c0 + j * L + iota
            crel = c - base
            m = (crel >= 0) & (crel < S)
            packed = (v << RSHIFT) | r
            plsc.store_scatter(winner, [jnp.clip(crel, 0, S - 1)], packed,
                               mask=m)
            return 0
        lax.fori_loop(0, CH // L, inner, 0)

    cstart(0, cca, cra, sem_ca)
    def wpair(g, _):
        t0 = 2 * g
        t1 = t0 + 1
        t2 = t0 + 2
        @pl.when(t1 < NCH)
        def _():
            cstart(t1, ccb, crb, sem_cb)
        cwait(t0, cca, cra, sem_ca)
        process(t0, cca, cra)
        @pl.when(t2 < NCH)
        def _():
            cstart(t2, cca, cra, sem_ca)
        @pl.when(t1 < NCH)
        def _():
            cwait(t1, ccb, crb, sem_cb)
            process(t1, ccb, crb)
        return 0
    lax.fori_loop(0, NPAIR, wpair, 0)

    big = jnp.full((L,), S, jnp.int32)
    def scan(i, carry):
        cnt, ffzvec = carry
        w = winner[pl.ds(i * L, L)]
        m = w >= 0
        idxv = i * L + iota
        plsc.store_compressed(cbig.at[pl.ds(cnt, L)], base + idxv, mask=m)
        plsc.store_compressed(rbig.at[pl.ds(cnt, L)], w & (RPAD - 1), mask=m)
        cnt = cnt + jnp.sum(m.astype(jnp.int32))
        ffzvec = jnp.minimum(ffzvec, jnp.where(m, big, idxv))
        return cnt, ffzvec
    cnt, ffzvec = lax.fori_loop(0, S // L, scan, (0, big))
    ffz = jnp.min(ffzvec)
    ffz = jnp.where(ffz >= S, 0, ffz)

    padc = (base + ffz) + jnp.zeros((L,), jnp.int32)
    padr = jnp.full((L,), RPAD, jnp.int32)
    tail0 = (cnt // L) * L
    def ptail(k, _):
        idxv = tail0 + k * L + iota
        m = (idxv >= cnt) & (idxv < S)
        safe = jnp.clip(idxv, 0, S - 1)
        plsc.store_scatter(cbig, [safe], padc, mask=m)
        plsc.store_scatter(rbig, [safe], padr, mask=m)
        return 0
    lax.fori_loop(0, K // L, ptail, 0)

    nf = (cnt + K - 1) // K

    def zdrain(k, _):
        pltpu.make_async_copy(zbuf, out_hbm.at[pl.ds(base + k * ZR, ZR)],
                              sem_z).wait()
        return 0
    lax.fori_loop(0, NZ, zdrain, 0)

    def idxcopy(f, cb, rb):
        def cp(k, _):
            cb[pl.ds(k * L, L)] = cbig[pl.ds(f * K + k * L, L)]
            rb[pl.ds(k * L, L)] = rbig[pl.ds(f * K + k * L, L)]
            return 0
        lax.fori_loop(0, K // L, cp, 0)

    def fpair(g, _):
        f0 = 2 * g
        f1 = f0 + 1
        @pl.when(f0 < nf)
        def _():
            idxcopy(f0, cb0, rb0)
            pltpu.async_copy(table_hbm.at[rb0], row0, sem_g0)
        @pl.when(f1 < nf)
        def _():
            idxcopy(f1, cb1, rb1)
            pltpu.async_copy(table_hbm.at[rb1], row1, sem_g1)
        @pl.when(f0 < nf)
        def _():
            pltpu.make_async_copy(table_hbm.at[rb0], row0, sem_g0).wait()
            pltpu.async_copy(row0, out_hbm.at[cb0], sem_s0)
        @pl.when(f1 < nf)
        def _():
            pltpu.make_async_copy(table_hbm.at[rb1], row1, sem_g1).wait()
            pltpu.async_copy(row1, out_hbm.at[cb1], sem_s1)
        @pl.when(f0 < nf)
        def _():
            pltpu.make_async_copy(row0, out_hbm.at[cb0], sem_s0).wait()
        @pl.when(f1 < nf)
        def _():
            pltpu.make_async_copy(row1, out_hbm.at[cb1], sem_s1).wait()
        return 0
    lax.fori_loop(0, (NFMAX + 1) // 2, fpair, 0)


@jax.jit
def _voxel_grid(cidx, rid, table):
    mesh = plsc.VectorSubcoreMesh(core_axis_name="c", subcore_axis_name="s",
                                  num_cores=NC, num_subcores=NS)
    f = pl.kernel(
        _body,
        out_type=jax.ShapeDtypeStruct((NROWS, FDIM), jnp.float32),
        mesh=mesh,
        compiler_params=pltpu.CompilerParams(needs_layout_passes=False),
        scratch_types=[
            pltpu.VMEM((ZR, FDIM), jnp.float32),
            pltpu.VMEM((S,), jnp.int32),
            pltpu.VMEM((S,), jnp.int32),
            pltpu.VMEM((S,), jnp.int32),
            pltpu.VMEM((CH,), jnp.int32),
            pltpu.VMEM((CH,), jnp.int32),
            pltpu.VMEM((CH,), jnp.int32),
            pltpu.VMEM((CH,), jnp.int32),
            pltpu.VMEM((K,), jnp.int32),
            pltpu.VMEM((K,), jnp.int32),
            pltpu.VMEM((K,), jnp.int32),
            pltpu.VMEM((K,), jnp.int32),
            pltpu.VMEM((K, FDIM), jnp.float32),
            pltpu.VMEM((K, FDIM), jnp.float32),
            pltpu.SemaphoreType.DMA,
            pltpu.SemaphoreType.DMA,
            pltpu.SemaphoreType.DMA,
            pltpu.SemaphoreType.DMA,
            pltpu.SemaphoreType.DMA,
            pltpu.SemaphoreType.DMA,
            pltpu.SemaphoreType.DMA,
        ],
    )
    return f(cidx, rid, table)


def kernel(voxels_argmax_centerIdx, voxels_argmax_batchResIds0Based,
           prot_feats0based, prot_multizProfiles, voxelFeats_proteinBatch):
    orig_shape = voxelFeats_proteinBatch.shape
    cidx = voxels_argmax_centerIdx.astype(jnp.int32)
    rid = voxels_argmax_batchResIds0Based.astype(jnp.int32)
    table = jnp.concatenate([prot_feats0based, prot_multizProfiles], axis=1)
    table = jnp.concatenate(
        [table, jnp.zeros((8, FDIM), jnp.float32)], axis=0)
    out = _voxel_grid(cidx, rid, table)
    return out.reshape(orig_shape)
